# baseline (speedup 1.0000x reference)
import dataclasses
import functools

import jax
import jax.numpy as jnp
import numpy as np
from jax import lax
from jax.experimental import pallas as pl
from jax.experimental.pallas import tpu as pltpu
from jax.experimental.pallas import tpu_sc as plsc

SIX = 6
B = 4096
F = 2048
M = SIX * F


_MBLK = 256
_BBLK = 512
_CHUNK = 16

_ROTS = ((13, 15, 26, 6), (17, 29, 16, 24))


def _rgen_body(key_ref, r_ref):
    mi = pl.program_id(0)
    bi = pl.program_id(1)
    k0 = key_ref[0]
    k1 = key_ref[1]
    ks2 = k0 ^ k1 ^ jnp.uint32(0x1BD11BDA)
    ks = (k0, k1, ks2)

    m0 = (mi * _MBLK).astype(jnp.uint32) + lax.broadcasted_iota(
        jnp.uint32, (_CHUNK, _BBLK), 0)
    b0 = (bi * _BBLK).astype(jnp.uint32) + lax.broadcasted_iota(
        jnp.uint32, (_CHUNK, _BBLK), 1)
    cnt0 = b0 * jnp.uint32(M) + m0 + k1

    def chunk(i, carry):
        x0 = ks[0]
        x1 = cnt0 + (i * _CHUNK).astype(jnp.uint32)
        for rnd in range(5):
            for r in _ROTS[rnd % 2]:
                x0 = x0 + x1
                x1 = (x1 << jnp.uint32(r)) | (x1 >> jnp.uint32(32 - r))
                x1 = x1 ^ x0
            x0 = x0 + ks[(rnd + 1) % 3]
            x1 = x1 + ks[(rnd + 2) % 3] + jnp.uint32(rnd + 1)
        bits = x0 ^ x1
        u = (bits >> jnp.uint32(9)) | jnp.uint32(0x3F800000)
        r_ref[pl.ds(i * _CHUNK, _CHUNK), :] = (
            lax.bitcast_convert_type(u, jnp.float32) - 1.0)
        return carry

    lax.fori_loop(0, _MBLK // _CHUNK, chunk, 0, unroll=4)


def _rgen(key2):
    return pl.pallas_call(
        _rgen_body,
        grid=(M // _MBLK, B // _BBLK),
        in_specs=[pl.BlockSpec(memory_space=pltpu.SMEM)],
        out_specs=pl.BlockSpec((_MBLK, _BBLK), lambda mi, bi: (mi, bi)),
        out_shape=jax.ShapeDtypeStruct((M, B), jnp.float32),
    )(key2)


def _sig_body(l_ref, o_ref):
    o_ref[...] = 1.0 / (1.0 + jnp.exp(-l_ref[...]))


def _sig_lut(lut):
    return pl.pallas_call(
        _sig_body, out_shape=jax.ShapeDtypeStruct(lut.shape, lut.dtype),
    )(lut)


def _affine_body(x_ref, o_ref):
    o_ref[...] = (x_ref[...] + 1.0) / 2.0


def _affine(xT):
    return pl.pallas_call(
        _affine_body, out_shape=jax.ShapeDtypeStruct(xT.shape, xT.dtype),
    )(xT)


_NW = 32
_FW = F // _NW
_G = 16
_R = _G * SIX
_BC = 256
_NB = B // _BC


def _make_sc_layer(fprev):
    mesh = plsc.VectorSubcoreMesh(core_axis_name="c", subcore_axis_name="s")
    cp = pltpu.CompilerParams()
    if "needs_layout_passes" in pltpu.CompilerParams.__dataclass_fields__:
        cp = dataclasses.replace(cp, needs_layout_passes=False)

    @functools.partial(
        pl.kernel,
        mesh=mesh,
        compiler_params=cp,
        out_type=jax.ShapeDtypeStruct((F, B), jnp.float32),
        scratch_types=[
            pltpu.VMEM((_FW * SIX,), jnp.int32),
            pltpu.VMEM((_FW * 64,), jnp.float32),
            pltpu.VMEM((_R,), jnp.int32),
            pltpu.VMEM((_R, _BC), jnp.float32),
            pltpu.VMEM((_R, _BC), jnp.float32),
            pltpu.VMEM((_G, _BC), jnp.float32),
            pltpu.SemaphoreType.DMA,
        ],
    )
    def sc_layer(x_hbm, r_hbm, conn_hbm, lut_hbm, y_hbm,
                 conn_v, lut_v, idx_v, g_v, r_v, y_v, sem):
        wid = lax.axis_index("s") * 2 + lax.axis_index("c")
        f0 = wid * _FW
        pltpu.sync_copy(conn_hbm.at[pl.ds(f0 * SIX, _FW * SIX)], conn_v)
        pltpu.sync_copy(lut_hbm.at[pl.ds(f0 * 64, _FW * 64)], lut_v)

        @pl.loop(0, _NB)
        def _(nb):
            b0 = nb * _BC
            for t in range(_FW // _G):
                for i in range(SIX):
                    c16 = conn_v[pl.ds(t * _R + i * 16, 16)]
                    idx_v[pl.ds(i * 16, 16)] = c16 * _NB + nb
                pltpu.async_copy(x_hbm.at[idx_v], g_v, sem).wait()
                pltpu.sync_copy(
                    r_hbm.at[pl.ds((f0 + t * _G) * SIX, _R), pl.ds(b0, _BC)],
                    r_v)

                @pl.loop(0, _G)
                def _(fl):
                    @pl.loop(0, _BC // 64)
                    def _(cc):
                        for u in range(4):
                            col = cc * 64 + u * 16
                            acc = jnp.zeros((16,), jnp.int32)
                            for j in range(SIX):
                                gv = g_v[fl * SIX + j, pl.ds(col, 16)]
                                rv = r_v[fl * SIX + j, pl.ds(col, 16)]
                                acc = acc + jnp.where(gv >= rv,
                                                      jnp.int32(1 << j),
                                                      jnp.int32(0))
                            val = plsc.load_gather(
                                lut_v, [acc + (t * _G + fl) * 64])
                            y_v[fl, pl.ds(col, 16)] = val

                pltpu.sync_copy(
                    y_v, y_hbm.at[pl.ds(f0 + t * _G, _G), pl.ds(b0, _BC)])

    def run(xT, r, conn, lut):
        x_flat = xT.reshape(fprev * _NB, _BC)
        return sc_layer(x_flat, r, conn.reshape(-1), lut.reshape(-1))

    return run


_make_sc_layer = functools.lru_cache(maxsize=None)(_make_sc_layer)


def kernel(inputs, lut1, lut2, lut3, connect_1, connect_2, connect_3):
    kd = jax.random.key_data(jax.random.split(jax.random.key(1234), 3))
    kd = kd.astype(jnp.uint32)
    xp = _affine(jnp.transpose(inputs))
    l1s = _sig_lut(lut1)
    l2s = _sig_lut(lut2)
    r1 = _rgen(kd[0])
    h1 = _make_sc_layer(784)(xp, r1, connect_1, l1s)
    r2 = _rgen(kd[1])
    h2 = _make_sc_layer(F)(h1, r2, connect_2, l2s)
    r3 = _rgen(kd[2])
    h3 = _make_sc_layer(F)(h2, r3, connect_3, lut3)
    return jnp.transpose(h3)


# speedup vs baseline: 1.1390x; 1.1390x over previous
import dataclasses
import functools

import jax
import jax.numpy as jnp
import numpy as np
from jax import lax
from jax.experimental import pallas as pl
from jax.experimental.pallas import tpu as pltpu
from jax.experimental.pallas import tpu_sc as plsc

SIX = 6
B = 4096
F = 2048
M = SIX * F


_MBLK = 256
_BBLK = 512
_CHUNK = 16

_ROTS = ((13, 15, 26, 6), (17, 29, 16, 24))


_BSTRIP = 1024
_NSTRIP = B // _BSTRIP


def _rgen_body(boff, key_ref, r_ref):
    mi = pl.program_id(0)
    bi = pl.program_id(1)
    k0 = key_ref[0]
    k1 = key_ref[1]
    ks2 = k0 ^ k1 ^ jnp.uint32(0x1BD11BDA)
    ks = (k0, k1, ks2)

    m0 = (mi * _MBLK).astype(jnp.uint32) + lax.broadcasted_iota(
        jnp.uint32, (_CHUNK, _BBLK), 0)
    b0 = jnp.uint32(boff) + (bi * _BBLK).astype(jnp.uint32) + lax.broadcasted_iota(
        jnp.uint32, (_CHUNK, _BBLK), 1)
    cnt0 = b0 * jnp.uint32(M) + m0 + k1

    def chunk(i, carry):
        x0 = ks[0]
        x1 = cnt0 + (i * _CHUNK).astype(jnp.uint32)
        for rnd in range(5):
            for r in _ROTS[rnd % 2]:
                x0 = x0 + x1
                x1 = (x1 << jnp.uint32(r)) | (x1 >> jnp.uint32(32 - r))
                x1 = x1 ^ x0
            x0 = x0 + ks[(rnd + 1) % 3]
            x1 = x1 + ks[(rnd + 2) % 3] + jnp.uint32(rnd + 1)
        bits = x0 ^ x1
        u = (bits >> jnp.uint32(9)) | jnp.uint32(0x3F800000)
        r_ref[pl.ds(i * _CHUNK, _CHUNK), :] = (
            lax.bitcast_convert_type(u, jnp.float32) - 1.0)
        return carry

    lax.fori_loop(0, _MBLK // _CHUNK, chunk, 0, unroll=4)


def _rgen(key2, boff):
    return pl.pallas_call(
        functools.partial(_rgen_body, boff),
        grid=(M // _MBLK, _BSTRIP // _BBLK),
        in_specs=[pl.BlockSpec(memory_space=pltpu.SMEM)],
        out_specs=pl.BlockSpec((_MBLK, _BBLK), lambda mi, bi: (mi, bi)),
        out_shape=jax.ShapeDtypeStruct((M, _BSTRIP), jnp.float32),
    )(key2)


def _sig_body(l_ref, o_ref):
    o_ref[...] = 1.0 / (1.0 + jnp.exp(-l_ref[...]))


def _sig_lut(lut):
    return pl.pallas_call(
        _sig_body, out_shape=jax.ShapeDtypeStruct(lut.shape, lut.dtype),
    )(lut)


_NW = 32
_FW = F // _NW
_G = 16
_R = _G * SIX
_BC = 256
_NBL = _BSTRIP // _BC


def _make_sc_layer(fprev, n_in, nb0, affine):
    mesh = plsc.VectorSubcoreMesh(core_axis_name="c", subcore_axis_name="s")
    cp = pltpu.CompilerParams()
    if "needs_layout_passes" in pltpu.CompilerParams.__dataclass_fields__:
        cp = dataclasses.replace(cp, needs_layout_passes=False)

    @functools.partial(
        pl.kernel,
        mesh=mesh,
        compiler_params=cp,
        out_type=jax.ShapeDtypeStruct((F, _BSTRIP), jnp.float32),
        scratch_types=[
            pltpu.VMEM((_FW * SIX,), jnp.int32),
            pltpu.VMEM((_FW * 64,), jnp.float32),
            pltpu.VMEM((_R,), jnp.int32),
            pltpu.VMEM((_R, _BC), jnp.float32),
            pltpu.VMEM((_R, _BC), jnp.float32),
            pltpu.VMEM((_G, _BC), jnp.float32),
            pltpu.SemaphoreType.DMA,
        ],
    )
    def sc_layer(x_hbm, r_hbm, conn_hbm, lut_hbm, y_hbm,
                 conn_v, lut_v, idx_v, g_v, r_v, y_v, sem):
        wid = lax.axis_index("s") * 2 + lax.axis_index("c")
        f0 = wid * _FW
        pltpu.sync_copy(conn_hbm.at[pl.ds(f0 * SIX, _FW * SIX)], conn_v)
        pltpu.sync_copy(lut_hbm.at[pl.ds(f0 * 64, _FW * 64)], lut_v)

        @pl.loop(0, _NBL)
        def _(nb):
            b0 = nb * _BC
            for t in range(_FW // _G):
                for i in range(SIX):
                    c16 = conn_v[pl.ds(t * _R + i * 16, 16)]
                    idx_v[pl.ds(i * 16, 16)] = c16 * n_in + (nb0 + nb)
                pltpu.async_copy(x_hbm.at[idx_v], g_v, sem).wait()
                pltpu.sync_copy(
                    r_hbm.at[pl.ds((f0 + t * _G) * SIX, _R), pl.ds(b0, _BC)],
                    r_v)

                @pl.loop(0, _G)
                def _(fl):
                    @pl.loop(0, _BC // 64)
                    def _(cc):
                        for u in range(4):
                            col = cc * 64 + u * 16
                            acc = jnp.zeros((16,), jnp.int32)
                            for j in range(SIX):
                                gv = g_v[fl * SIX + j, pl.ds(col, 16)]
                                if affine:
                                    gv = (gv + 1.0) / 2.0
                                rv = r_v[fl * SIX + j, pl.ds(col, 16)]
                                acc = acc + jnp.where(gv >= rv,
                                                      jnp.int32(1 << j),
                                                      jnp.int32(0))
                            val = plsc.load_gather(
                                lut_v, [acc + (t * _G + fl) * 64])
                            y_v[fl, pl.ds(col, 16)] = val

                pltpu.sync_copy(
                    y_v, y_hbm.at[pl.ds(f0 + t * _G, _G), pl.ds(b0, _BC)])

    def run(x_flat, r, conn, lut):
        return sc_layer(x_flat, r, conn.reshape(-1), lut.reshape(-1))

    return run


_make_sc_layer = functools.lru_cache(maxsize=None)(_make_sc_layer)


def kernel(inputs, lut1, lut2, lut3, connect_1, connect_2, connect_3):
    kd = jax.random.key_data(jax.random.split(jax.random.key(1234), 3))
    kd = kd.astype(jnp.uint32)
    nb_full = B // _BC
    x_flat = jnp.transpose(inputs).reshape(784 * nb_full, _BC)
    l1s = _sig_lut(lut1)
    l2s = _sig_lut(lut2)
    outs = []
    for s in range(_NSTRIP):
        r1 = _rgen(kd[0], s * _BSTRIP)
        h1 = _make_sc_layer(784, nb_full, s * _NBL, True)(
            x_flat, r1, connect_1, l1s)
        r2 = _rgen(kd[1], s * _BSTRIP)
        h2 = _make_sc_layer(F, _NBL, 0, False)(
            h1.reshape(F * _NBL, _BC), r2, connect_2, l2s)
        r3 = _rgen(kd[2], s * _BSTRIP)
        h3 = _make_sc_layer(F, _NBL, 0, False)(
            h2.reshape(F * _NBL, _BC), r3, connect_3, lut3)
        outs.append(jnp.transpose(h3))
    return jnp.concatenate(outs, axis=0)


# speedup vs baseline: 1.1659x; 1.0236x over previous
import dataclasses
import functools

import jax
import jax.numpy as jnp
import numpy as np
from jax import lax
from jax.experimental import pallas as pl
from jax.experimental.pallas import tpu as pltpu
from jax.experimental.pallas import tpu_sc as plsc

SIX = 6
B = 4096
F = 2048
M = SIX * F


_MBLK = 256
_BBLK = 512
_CHUNK = 16

_ROTS = ((13, 15, 26, 6), (17, 29, 16, 24))


_BSTRIP = 512
_NSTRIP = B // _BSTRIP


def _rgen_body(boff, key_ref, r_ref):
    mi = pl.program_id(0)
    bi = pl.program_id(1)
    k0 = key_ref[0]
    k1 = key_ref[1]
    ks2 = k0 ^ k1 ^ jnp.uint32(0x1BD11BDA)
    ks = (k0, k1, ks2)

    m0 = (mi * _MBLK).astype(jnp.uint32) + lax.broadcasted_iota(
        jnp.uint32, (_CHUNK, _BBLK), 0)
    b0 = jnp.uint32(boff) + (bi * _BBLK).astype(jnp.uint32) + lax.broadcasted_iota(
        jnp.uint32, (_CHUNK, _BBLK), 1)
    cnt0 = b0 * jnp.uint32(M) + m0 + k1

    def chunk(i, carry):
        x0 = ks[0]
        x1 = cnt0 + (i * _CHUNK).astype(jnp.uint32)
        for rnd in range(5):
            for r in _ROTS[rnd % 2]:
                x0 = x0 + x1
                x1 = (x1 << jnp.uint32(r)) | (x1 >> jnp.uint32(32 - r))
                x1 = x1 ^ x0
            x0 = x0 + ks[(rnd + 1) % 3]
            x1 = x1 + ks[(rnd + 2) % 3] + jnp.uint32(rnd + 1)
        bits = x0 ^ x1
        u = (bits >> jnp.uint32(9)) | jnp.uint32(0x3F800000)
        r_ref[pl.ds(i * _CHUNK, _CHUNK), :] = (
            lax.bitcast_convert_type(u, jnp.float32) - 1.0)
        return carry

    lax.fori_loop(0, _MBLK // _CHUNK, chunk, 0, unroll=4)


def _rgen(key2, boff):
    return pl.pallas_call(
        functools.partial(_rgen_body, boff),
        grid=(M // _MBLK, _BSTRIP // _BBLK),
        in_specs=[pl.BlockSpec(memory_space=pltpu.SMEM)],
        out_specs=pl.BlockSpec((_MBLK, _BBLK), lambda mi, bi: (mi, bi)),
        out_shape=jax.ShapeDtypeStruct((M, _BSTRIP), jnp.float32),
    )(key2)


def _sig_body(l_ref, o_ref):
    o_ref[...] = 1.0 / (1.0 + jnp.exp(-l_ref[...]))


def _sig_lut(lut):
    return pl.pallas_call(
        _sig_body, out_shape=jax.ShapeDtypeStruct(lut.shape, lut.dtype),
    )(lut)


_NW = 32
_FW = F // _NW
_G = 16
_R = _G * SIX
_BC = 256
_NBL = _BSTRIP // _BC


def _make_sc_layer(fprev, n_in, nb0, affine):
    mesh = plsc.VectorSubcoreMesh(core_axis_name="c", subcore_axis_name="s")
    cp = pltpu.CompilerParams()
    if "needs_layout_passes" in pltpu.CompilerParams.__dataclass_fields__:
        cp = dataclasses.replace(cp, needs_layout_passes=False)

    @functools.partial(
        pl.kernel,
        mesh=mesh,
        compiler_params=cp,
        out_type=jax.ShapeDtypeStruct((F, _BSTRIP), jnp.float32),
        scratch_types=[
            pltpu.VMEM((_FW * SIX,), jnp.int32),
            pltpu.VMEM((_FW * 64,), jnp.float32),
            pltpu.VMEM((_R,), jnp.int32),
            pltpu.VMEM((_R, _BC), jnp.float32),
            pltpu.VMEM((_R, _BC), jnp.float32),
            pltpu.VMEM((_G, _BC), jnp.float32),
            pltpu.SemaphoreType.DMA,
        ],
    )
    def sc_layer(x_hbm, r_hbm, conn_hbm, lut_hbm, y_hbm,
                 conn_v, lut_v, idx_v, g_v, r_v, y_v, sem):
        wid = lax.axis_index("s") * 2 + lax.axis_index("c")
        f0 = wid * _FW
        pltpu.sync_copy(conn_hbm.at[pl.ds(f0 * SIX, _FW * SIX)], conn_v)
        pltpu.sync_copy(lut_hbm.at[pl.ds(f0 * 64, _FW * 64)], lut_v)

        @pl.loop(0, _NBL)
        def _(nb):
            b0 = nb * _BC
            for t in range(_FW // _G):
                for i in range(SIX):
                    c16 = conn_v[pl.ds(t * _R + i * 16, 16)]
                    idx_v[pl.ds(i * 16, 16)] = c16 * n_in + (nb0 + nb)
                pltpu.async_copy(x_hbm.at[idx_v], g_v, sem).wait()
                pltpu.sync_copy(
                    r_hbm.at[pl.ds((f0 + t * _G) * SIX, _R), pl.ds(b0, _BC)],
                    r_v)

                @pl.loop(0, _G)
                def _(fl):
                    @pl.loop(0, _BC // 64)
                    def _(cc):
                        for u in range(4):
                            col = cc * 64 + u * 16
                            acc = jnp.zeros((16,), jnp.int32)
                            for j in range(SIX):
                                gv = g_v[fl * SIX + j, pl.ds(col, 16)]
                                if affine:
                                    gv = (gv + 1.0) / 2.0
                                rv = r_v[fl * SIX + j, pl.ds(col, 16)]
                                acc = acc + jnp.where(gv >= rv,
                                                      jnp.int32(1 << j),
                                                      jnp.int32(0))
                            val = plsc.load_gather(
                                lut_v, [acc + (t * _G + fl) * 64])
                            y_v[fl, pl.ds(col, 16)] = val

                pltpu.sync_copy(
                    y_v, y_hbm.at[pl.ds(f0 + t * _G, _G), pl.ds(b0, _BC)])

    def run(x_flat, r, conn, lut):
        return sc_layer(x_flat, r, conn.reshape(-1), lut.reshape(-1))

    return run


_make_sc_layer = functools.lru_cache(maxsize=None)(_make_sc_layer)


def kernel(inputs, lut1, lut2, lut3, connect_1, connect_2, connect_3):
    kd = jax.random.key_data(jax.random.split(jax.random.key(1234), 3))
    kd = kd.astype(jnp.uint32)
    nb_full = B // _BC
    x_flat = jnp.transpose(inputs).reshape(784 * nb_full, _BC)
    l1s = _sig_lut(lut1)
    l2s = _sig_lut(lut2)
    outs = []
    for s in range(_NSTRIP):
        r1 = _rgen(kd[0], s * _BSTRIP)
        h1 = _make_sc_layer(784, nb_full, s * _NBL, True)(
            x_flat, r1, connect_1, l1s)
        r2 = _rgen(kd[1], s * _BSTRIP)
        h2 = _make_sc_layer(F, _NBL, 0, False)(
            h1.reshape(F * _NBL, _BC), r2, connect_2, l2s)
        r3 = _rgen(kd[2], s * _BSTRIP)
        h3 = _make_sc_layer(F, _NBL, 0, False)(
            h2.reshape(F * _NBL, _BC), r3, connect_3, lut3)
        outs.append(jnp.transpose(h3))
    return jnp.concatenate(outs, axis=0)


# speedup vs baseline: 1.1803x; 1.0123x over previous
import dataclasses
import functools

import jax
import jax.numpy as jnp
import numpy as np
from jax import lax
from jax.experimental import pallas as pl
from jax.experimental.pallas import tpu as pltpu
from jax.experimental.pallas import tpu_sc as plsc

SIX = 6
B = 4096
F = 2048
M = SIX * F


_MBLK = 256
_BBLK = 256
_CHUNK = 16

_ROTS = ((13, 15, 26, 6), (17, 29, 16, 24))


_BSTRIP = 256
_NSTRIP = B // _BSTRIP


def _rgen_body(boff, key_ref, r_ref):
    mi = pl.program_id(0)
    bi = pl.program_id(1)
    k0 = key_ref[0]
    k1 = key_ref[1]
    ks2 = k0 ^ k1 ^ jnp.uint32(0x1BD11BDA)
    ks = (k0, k1, ks2)

    m0 = (mi * _MBLK).astype(jnp.uint32) + lax.broadcasted_iota(
        jnp.uint32, (_CHUNK, _BBLK), 0)
    b0 = jnp.uint32(boff) + (bi * _BBLK).astype(jnp.uint32) + lax.broadcasted_iota(
        jnp.uint32, (_CHUNK, _BBLK), 1)
    cnt0 = b0 * jnp.uint32(M) + m0 + k1

    def chunk(i, carry):
        x0 = ks[0]
        x1 = cnt0 + (i * _CHUNK).astype(jnp.uint32)
        for rnd in range(5):
            for r in _ROTS[rnd % 2]:
                x0 = x0 + x1
                x1 = (x1 << jnp.uint32(r)) | (x1 >> jnp.uint32(32 - r))
                x1 = x1 ^ x0
            x0 = x0 + ks[(rnd + 1) % 3]
            x1 = x1 + ks[(rnd + 2) % 3] + jnp.uint32(rnd + 1)
        bits = x0 ^ x1
        u = (bits >> jnp.uint32(9)) | jnp.uint32(0x3F800000)
        r_ref[pl.ds(i * _CHUNK, _CHUNK), :] = (
            lax.bitcast_convert_type(u, jnp.float32) - 1.0)
        return carry

    lax.fori_loop(0, _MBLK // _CHUNK, chunk, 0, unroll=4)


def _rgen(key2, boff):
    return pl.pallas_call(
        functools.partial(_rgen_body, boff),
        grid=(M // _MBLK, _BSTRIP // _BBLK),
        in_specs=[pl.BlockSpec(memory_space=pltpu.SMEM)],
        out_specs=pl.BlockSpec((_MBLK, _BBLK), lambda mi, bi: (mi, bi)),
        out_shape=jax.ShapeDtypeStruct((M, _BSTRIP), jnp.float32),
    )(key2)


def _sig_body(l_ref, o_ref):
    o_ref[...] = 1.0 / (1.0 + jnp.exp(-l_ref[...]))


def _sig_lut(lut):
    return pl.pallas_call(
        _sig_body, out_shape=jax.ShapeDtypeStruct(lut.shape, lut.dtype),
    )(lut)


_NW = 32
_FW = F // _NW
_G = 16
_R = _G * SIX
_BC = 256
_NBL = _BSTRIP // _BC


def _make_sc_layer(fprev, n_in, nb0, affine):
    mesh = plsc.VectorSubcoreMesh(core_axis_name="c", subcore_axis_name="s")
    cp = pltpu.CompilerParams()
    if "needs_layout_passes" in pltpu.CompilerParams.__dataclass_fields__:
        cp = dataclasses.replace(cp, needs_layout_passes=False)

    @functools.partial(
        pl.kernel,
        mesh=mesh,
        compiler_params=cp,
        out_type=jax.ShapeDtypeStruct((F, _BSTRIP), jnp.float32),
        scratch_types=[
            pltpu.VMEM((_FW * SIX,), jnp.int32),
            pltpu.VMEM((_FW * 64,), jnp.float32),
            pltpu.VMEM((_R,), jnp.int32),
            pltpu.VMEM((_R, _BC), jnp.float32),
            pltpu.VMEM((_R, _BC), jnp.float32),
            pltpu.VMEM((_G, _BC), jnp.float32),
            pltpu.SemaphoreType.DMA,
        ],
    )
    def sc_layer(x_hbm, r_hbm, conn_hbm, lut_hbm, y_hbm,
                 conn_v, lut_v, idx_v, g_v, r_v, y_v, sem):
        wid = lax.axis_index("s") * 2 + lax.axis_index("c")
        f0 = wid * _FW
        pltpu.sync_copy(conn_hbm.at[pl.ds(f0 * SIX, _FW * SIX)], conn_v)
        pltpu.sync_copy(lut_hbm.at[pl.ds(f0 * 64, _FW * 64)], lut_v)

        @pl.loop(0, _NBL)
        def _(nb):
            b0 = nb * _BC
            for t in range(_FW // _G):
                for i in range(SIX):
                    c16 = conn_v[pl.ds(t * _R + i * 16, 16)]
                    idx_v[pl.ds(i * 16, 16)] = c16 * n_in + (nb0 + nb)
                pltpu.async_copy(x_hbm.at[idx_v], g_v, sem).wait()
                pltpu.sync_copy(
                    r_hbm.at[pl.ds((f0 + t * _G) * SIX, _R), pl.ds(b0, _BC)],
                    r_v)

                @pl.loop(0, _G)
                def _(fl):
                    @pl.loop(0, _BC // 64)
                    def _(cc):
                        for u in range(4):
                            col = cc * 64 + u * 16
                            acc = jnp.zeros((16,), jnp.int32)
                            for j in range(SIX):
                                gv = g_v[fl * SIX + j, pl.ds(col, 16)]
                                if affine:
                                    gv = (gv + 1.0) / 2.0
                                rv = r_v[fl * SIX + j, pl.ds(col, 16)]
                                acc = acc + jnp.where(gv >= rv,
                                                      jnp.int32(1 << j),
                                                      jnp.int32(0))
                            val = plsc.load_gather(
                                lut_v, [acc + (t * _G + fl) * 64])
                            y_v[fl, pl.ds(col, 16)] = val

                pltpu.sync_copy(
                    y_v, y_hbm.at[pl.ds(f0 + t * _G, _G), pl.ds(b0, _BC)])

    def run(x_flat, r, conn, lut):
        return sc_layer(x_flat, r, conn.reshape(-1), lut.reshape(-1))

    return run


_make_sc_layer = functools.lru_cache(maxsize=None)(_make_sc_layer)


def kernel(inputs, lut1, lut2, lut3, connect_1, connect_2, connect_3):
    kd = jax.random.key_data(jax.random.split(jax.random.key(1234), 3))
    kd = kd.astype(jnp.uint32)
    nb_full = B // _BC
    x_flat = jnp.transpose(inputs).reshape(784 * nb_full, _BC)
    l1s = _sig_lut(lut1)
    l2s = _sig_lut(lut2)
    outs = []
    for s in range(_NSTRIP):
        r1 = _rgen(kd[0], s * _BSTRIP)
        h1 = _make_sc_layer(784, nb_full, s * _NBL, True)(
            x_flat, r1, connect_1, l1s)
        r2 = _rgen(kd[1], s * _BSTRIP)
        h2 = _make_sc_layer(F, _NBL, 0, False)(
            h1.reshape(F * _NBL, _BC), r2, connect_2, l2s)
        r3 = _rgen(kd[2], s * _BSTRIP)
        h3 = _make_sc_layer(F, _NBL, 0, False)(
            h2.reshape(F * _NBL, _BC), r3, connect_3, lut3)
        outs.append(jnp.transpose(h3))
    return jnp.concatenate(outs, axis=0)
